# baseline (speedup 1.0000x reference)
import functools

import jax
import jax.numpy as jnp
from jax import lax
from jax.experimental import pallas as pl
from jax.experimental.pallas import tpu as pltpu
from jax.experimental.pallas import tpu_sc as plsc

_MAX_LEN = 8192
_D = 1024
_B = 4
_NC = 2
_NS = 16
_NW = _NC * _NS
_ROWS = _MAX_LEN // _NW
_CHUNK = 64
_NCHUNK = _ROWS // _CHUNK
_B_SC = 3
_B_TC = _B - _B_SC

_mesh = plsc.VectorSubcoreMesh(core_axis_name="c", subcore_axis_name="s")


@functools.partial(
    pl.kernel,
    mesh=_mesh,
    out_type=jax.ShapeDtypeStruct((_B_SC * _MAX_LEN, _D), jnp.float32),
    scratch_types=[pltpu.VMEM((_CHUNK, _D), jnp.float32)],
)
def _bcast_sc(pe_hbm, out_hbm, buf):
    wid = lax.axis_index("s") * _NC + lax.axis_index("c")
    base = wid * _ROWS
    for i in range(_NCHUNK):
        r0 = base + i * _CHUNK
        pltpu.sync_copy(pe_hbm.at[pl.ds(r0, _CHUNK)], buf)
        for b in range(_B_SC):
            pltpu.sync_copy(buf, out_hbm.at[pl.ds(b * _MAX_LEN + r0, _CHUNK)])


_TC_BLK = 1024


def _copy_body(pe_ref, out_ref):
    out_ref[...] = pe_ref[...]


_bcast_tc = pl.pallas_call(
    _copy_body,
    grid=(_MAX_LEN // _TC_BLK,),
    in_specs=[pl.BlockSpec((_TC_BLK, _D), lambda j: (j, 0))],
    out_specs=pl.BlockSpec((_TC_BLK, _D), lambda j: (j, 0)),
    out_shape=jax.ShapeDtypeStruct((_MAX_LEN, _D), jnp.float32),
)


def kernel(x, pe_table):
    del x
    sc_out = _bcast_sc(pe_table).reshape(_B_SC, _MAX_LEN, _D)
    tc_out = _bcast_tc(pe_table).reshape(_B_TC, _MAX_LEN, _D)
    return jnp.concatenate([tc_out, sc_out], axis=0)


# speedup vs baseline: 2.2588x; 2.2588x over previous
import functools

import jax
import jax.numpy as jnp
from jax import lax
from jax.experimental import pallas as pl
from jax.experimental.pallas import tpu as pltpu
from jax.experimental.pallas import tpu_sc as plsc

_MAX_LEN = 8192
_D = 1024
_B = 4
_NC = 2
_NS = 16
_NW = _NC * _NS
_ROWS = _MAX_LEN // _NW
_CHUNK = 64
_NCHUNK = _ROWS // _CHUNK

_mesh = plsc.VectorSubcoreMesh(core_axis_name="c", subcore_axis_name="s")


@functools.partial(
    pl.kernel,
    mesh=_mesh,
    out_type=jax.ShapeDtypeStruct((_B * _MAX_LEN, _D), jnp.float32),
    scratch_types=[pltpu.VMEM((_CHUNK, _D), jnp.float32)],
)
def _bcast(pe_hbm, out_hbm, buf):
    wid = lax.axis_index("s") * _NC + lax.axis_index("c")
    base = wid * _ROWS
    for i in range(_NCHUNK):
        r0 = base + i * _CHUNK
        pltpu.sync_copy(pe_hbm.at[pl.ds(r0, _CHUNK)], buf)
        for b in range(_B):
            pltpu.sync_copy(buf, out_hbm.at[pl.ds(b * _MAX_LEN + r0, _CHUNK)])


def kernel(x, pe_table):
    del x
    return _bcast(pe_table).reshape(_B, _MAX_LEN, _D)
